# Initial kernel scaffold; baseline (speedup 1.0000x reference)
import jax
import jax.numpy as jnp
from jax.experimental import pallas as pl


def kernel(x, edge_index, edge_type, basis, att, root, bias):
    raise NotImplementedError("write your pallas kernel here")


# speedup vs baseline: 17.1909x; 17.1909x over previous
import functools

import jax
import jax.numpy as jnp
from jax import lax
from jax.experimental import pallas as pl
from jax.experimental.pallas import tpu as pltpu
from jax.experimental.pallas import tpu_sc as plsc

N_NODES = 10000
D = 128
N_EDGES = 320000
N_REL = 8
N_BASES = 4

NC = 2
NS = 16
LANES = 16
NW = NC * NS
EDGES_PER_W = N_EDGES // NW
CHUNK = 128
FULL_CHUNKS = EDGES_PER_W // CHUNK
TAIL = EDGES_PER_W - FULL_CHUNKS * CHUNK
ROWS_PER_SUB = 624
ROWS_TAIL = N_NODES - NS * ROWS_PER_SUB


def _xw_body(att_ref, x_ref, basis_ref, out_ref):
    r = pl.program_id(0)
    w = (att_ref[r, 0] * basis_ref[0] + att_ref[r, 1] * basis_ref[1]
         + att_ref[r, 2] * basis_ref[2] + att_ref[r, 3] * basis_ref[3])
    out_ref[...] = jnp.dot(x_ref[...], w, preferred_element_type=jnp.float32)


def _xw_table(att, x, basis):
    return pl.pallas_call(
        _xw_body,
        grid=(N_REL,),
        in_specs=[
            pl.BlockSpec(memory_space=pltpu.SMEM),
            pl.BlockSpec((N_NODES, D), lambda r: (0, 0)),
            pl.BlockSpec((N_BASES, D, D), lambda r: (0, 0, 0)),
        ],
        out_specs=pl.BlockSpec((N_NODES, D), lambda r: (r, 0)),
        out_shape=jax.ShapeDtypeStruct((N_REL * N_NODES, D), jnp.float32),
    )(att, x, basis)


_MESH = plsc.VectorSubcoreMesh(core_axis_name="c", subcore_axis_name="s")


@functools.partial(
    pl.kernel,
    mesh=_MESH,
    out_type=jax.ShapeDtypeStruct((NC * N_NODES, D), jnp.float32),
    scratch_types=[
        pltpu.VMEM((CHUNK,), jnp.int32),
        pltpu.VMEM((CHUNK,), jnp.int32),
        pltpu.VMEM((CHUNK,), jnp.int32),
        pltpu.VMEM((CHUNK,), jnp.int32),
        pltpu.VMEM((CHUNK, D), jnp.float32),
        pltpu.VMEM((TAIL,), jnp.int32),
        pltpu.VMEM((TAIL,), jnp.int32),
        pltpu.VMEM((TAIL,), jnp.int32),
        pltpu.VMEM((TAIL,), jnp.int32),
        pltpu.VMEM((TAIL, D), jnp.float32),
        pltpu.VMEM_SHARED((N_NODES, D), jnp.float32),
        pltpu.SemaphoreType.DMA,
    ],
)
def _sc_aggregate(xw_hbm, src_hbm, et_hbm, dst_hbm, zeros_hbm, out_hbm,
                  src_v, et_v, dst_v, idx_v, rows_v,
                  src_t, et_t, dst_t, idx_t, rows_t,
                  acc_sh, sem):
    c = lax.axis_index("c")
    s = lax.axis_index("s")
    wid = c * NS + s

    pltpu.sync_copy(zeros_hbm,
                    acc_sh.at[pl.ds(s * ROWS_PER_SUB, ROWS_PER_SUB)])

    @pl.when(s == NS - 1)
    def _zero_tail():
        pltpu.sync_copy(zeros_hbm.at[pl.ds(0, ROWS_TAIL)],
                        acc_sh.at[pl.ds(NS * ROWS_PER_SUB, ROWS_TAIL)])

    plsc.subcore_barrier()

    base_w = wid * EDGES_PER_W

    def do_chunk(base, size, src_b, et_b, dst_b, idx_b, rows_b):
        pltpu.sync_copy(src_hbm.at[pl.ds(base, size)], src_b)
        pltpu.sync_copy(et_hbm.at[pl.ds(base, size)], et_b)
        pltpu.sync_copy(dst_hbm.at[pl.ds(base, size)], dst_b)
        for i in range(size // LANES):
            sl = pl.ds(i * LANES, LANES)
            idx_b[sl] = et_b[sl] * N_NODES + src_b[sl]
        pltpu.async_copy(xw_hbm.at[idx_b], rows_b, sem).wait()
        pltpu.sync_copy(rows_b, acc_sh.at[dst_b], add=True)

    def g_body(g, carry):
        do_chunk(base_w + g * CHUNK, CHUNK, src_v, et_v, dst_v, idx_v, rows_v)
        return carry

    lax.fori_loop(0, FULL_CHUNKS, g_body, 0)
    do_chunk(base_w + FULL_CHUNKS * CHUNK, TAIL,
             src_t, et_t, dst_t, idx_t, rows_t)

    plsc.subcore_barrier()
    row0 = s * ROWS_PER_SUB
    pltpu.sync_copy(acc_sh.at[pl.ds(row0, ROWS_PER_SUB)],
                    out_hbm.at[pl.ds(c * N_NODES + row0, ROWS_PER_SUB)])

    @pl.when(s == NS - 1)
    def _write_tail():
        t0 = NS * ROWS_PER_SUB
        pltpu.sync_copy(acc_sh.at[pl.ds(t0, ROWS_TAIL)],
                        out_hbm.at[pl.ds(c * N_NODES + t0, ROWS_TAIL)])


_CBLK = 2000


def _combine_body(x_ref, root_ref, bias_ref, p0_ref, p1_ref, out_ref):
    out_ref[...] = (
        jnp.dot(x_ref[...], root_ref[...], preferred_element_type=jnp.float32)
        + bias_ref[...] + p0_ref[...] + p1_ref[...])


def _combine(x, root, bias2d, partials):
    nblk = N_NODES // _CBLK
    return pl.pallas_call(
        _combine_body,
        grid=(nblk,),
        in_specs=[
            pl.BlockSpec((_CBLK, D), lambda i: (i, 0)),
            pl.BlockSpec((D, D), lambda i: (0, 0)),
            pl.BlockSpec((1, D), lambda i: (0, 0)),
            pl.BlockSpec((_CBLK, D), lambda i: (i, 0)),
            pl.BlockSpec((_CBLK, D), lambda i, _n=nblk: (i + _n, 0)),
        ],
        out_specs=pl.BlockSpec((_CBLK, D), lambda i: (i, 0)),
        out_shape=jax.ShapeDtypeStruct((N_NODES, D), jnp.float32),
    )(x, root, bias2d, partials, partials)


def kernel(x, edge_index, edge_type, basis, att, root, bias):
    src = edge_index[0].astype(jnp.int32)
    dst = edge_index[1].astype(jnp.int32)
    et = edge_type.astype(jnp.int32)
    xw = _xw_table(att, x, basis)
    zeros = jnp.zeros((ROWS_PER_SUB, D), jnp.float32)
    partials = _sc_aggregate(xw, src, et, dst, zeros)
    return _combine(x, root, bias.reshape(1, D), partials)


# speedup vs baseline: 33.6429x; 1.9570x over previous
import functools

import jax
import jax.numpy as jnp
from jax import lax
from jax.experimental import pallas as pl
from jax.experimental.pallas import tpu as pltpu
from jax.experimental.pallas import tpu_sc as plsc

N_NODES = 10000
D = 128
N_EDGES = 320000
N_REL = 8
N_BASES = 4

NC = 2
NS = 16
LANES = 16
NW = NC * NS
EDGES_PER_W = N_EDGES // NW
CHUNK = 96
FULL_CHUNKS = EDGES_PER_W // CHUNK
TAIL = EDGES_PER_W - FULL_CHUNKS * CHUNK
ROWS_PER_SUB = 624
ROWS_TAIL = N_NODES - NS * ROWS_PER_SUB


def _xw_body(att_ref, x_ref, basis_ref, out_ref):
    r = pl.program_id(0)
    w = (att_ref[r, 0] * basis_ref[0] + att_ref[r, 1] * basis_ref[1]
         + att_ref[r, 2] * basis_ref[2] + att_ref[r, 3] * basis_ref[3])
    out_ref[...] = jnp.dot(x_ref[...], w, preferred_element_type=jnp.float32)


def _xw_table(att, x, basis):
    return pl.pallas_call(
        _xw_body,
        grid=(N_REL,),
        in_specs=[
            pl.BlockSpec(memory_space=pltpu.SMEM),
            pl.BlockSpec((N_NODES, D), lambda r: (0, 0)),
            pl.BlockSpec((N_BASES, D, D), lambda r: (0, 0, 0)),
        ],
        out_specs=pl.BlockSpec((N_NODES, D), lambda r: (r, 0)),
        out_shape=jax.ShapeDtypeStruct((N_REL * N_NODES, D), jnp.float32),
    )(att, x, basis)


def _flat_idx_body(et_ref, src_ref, out_ref):
    out_ref[...] = et_ref[...] * N_NODES + src_ref[...]


def _flat_idx(et2d, src2d):
    rows = N_EDGES // D
    return pl.pallas_call(
        _flat_idx_body,
        grid=(1,),
        in_specs=[
            pl.BlockSpec((rows, D), lambda i: (0, 0)),
            pl.BlockSpec((rows, D), lambda i: (0, 0)),
        ],
        out_specs=pl.BlockSpec((rows, D), lambda i: (0, 0)),
        out_shape=jax.ShapeDtypeStruct((rows, D), jnp.int32),
    )(et2d, src2d)


_MESH = plsc.VectorSubcoreMesh(core_axis_name="c", subcore_axis_name="s")


@functools.partial(
    pl.kernel,
    mesh=_MESH,
    out_type=jax.ShapeDtypeStruct((NC * N_NODES, D), jnp.float32),
    scratch_types=[
        pltpu.VMEM((EDGES_PER_W,), jnp.int32),
        pltpu.VMEM((EDGES_PER_W,), jnp.int32),
        pltpu.VMEM((CHUNK,), jnp.int32),
        pltpu.VMEM((CHUNK,), jnp.int32),
        pltpu.VMEM((CHUNK, D), jnp.float32),
        pltpu.VMEM((CHUNK, D), jnp.float32),
        pltpu.VMEM((TAIL,), jnp.int32),
        pltpu.VMEM((TAIL, D), jnp.float32),
        pltpu.VMEM_SHARED((N_NODES, D), jnp.float32),
        pltpu.SemaphoreType.DMA,
        pltpu.SemaphoreType.DMA,
    ],
)
def _sc_aggregate(xw_hbm, idx_hbm, dst_hbm, zeros_hbm, out_hbm,
                  idx_all, dst_all,
                  dstg0, dstg1, rows0, rows1, dstgt, rows_t,
                  acc_sh, sem0, sem1):
    c = lax.axis_index("c")
    s = lax.axis_index("s")
    wid = c * NS + s
    base_w = wid * EDGES_PER_W

    pltpu.sync_copy(idx_hbm.at[pl.ds(base_w, EDGES_PER_W)], idx_all)
    pltpu.sync_copy(dst_hbm.at[pl.ds(base_w, EDGES_PER_W)], dst_all)

    pltpu.sync_copy(zeros_hbm,
                    acc_sh.at[pl.ds(s * ROWS_PER_SUB, ROWS_PER_SUB)])

    @pl.when(s == NS - 1)
    def _zero_tail():
        pltpu.sync_copy(zeros_hbm.at[pl.ds(0, ROWS_TAIL)],
                        acc_sh.at[pl.ds(NS * ROWS_PER_SUB, ROWS_TAIL)])

    plsc.subcore_barrier()

    def fire(g, rows_b, sem_b):
        pltpu.async_copy(xw_hbm.at[idx_all.at[pl.ds(g * CHUNK, CHUNK)]],
                         rows_b, sem_b)

    def drain(g, rows_b, sem_b):
        pltpu.make_async_copy(xw_hbm.at[idx_all.at[pl.ds(g * CHUNK, CHUNK)]],
                              rows_b, sem_b).wait()

    def scatter(g, dstg_b, rows_b):
        for j in range(CHUNK // LANES):
            sl = pl.ds(j * LANES, LANES)
            dstg_b[sl] = dst_all[pl.ds(g * CHUNK + j * LANES, LANES)]
        pltpu.sync_copy(rows_b, acc_sh.at[dstg_b], add=True)

    fire(0, rows0, sem0)

    def pipe_body(i, carry):
        g0 = 2 * i
        fire(g0 + 1, rows1, sem1)
        drain(g0, rows0, sem0)
        scatter(g0, dstg0, rows0)

        @pl.when(i < FULL_CHUNKS // 2 - 1)
        def _refire():
            fire(g0 + 2, rows0, sem0)

        drain(g0 + 1, rows1, sem1)
        scatter(g0 + 1, dstg1, rows1)
        return carry

    lax.fori_loop(0, FULL_CHUNKS // 2, pipe_body, 0)

    t0 = FULL_CHUNKS * CHUNK
    pltpu.async_copy(xw_hbm.at[idx_all.at[pl.ds(t0, TAIL)]],
                     rows_t, sem0).wait()
    dstgt[...] = dst_all[pl.ds(t0, TAIL)]
    pltpu.sync_copy(rows_t, acc_sh.at[dstgt], add=True)

    plsc.subcore_barrier()
    row0 = s * ROWS_PER_SUB
    pltpu.sync_copy(acc_sh.at[pl.ds(row0, ROWS_PER_SUB)],
                    out_hbm.at[pl.ds(c * N_NODES + row0, ROWS_PER_SUB)])

    @pl.when(s == NS - 1)
    def _write_tail():
        t0 = NS * ROWS_PER_SUB
        pltpu.sync_copy(acc_sh.at[pl.ds(t0, ROWS_TAIL)],
                        out_hbm.at[pl.ds(c * N_NODES + t0, ROWS_TAIL)])


_CBLK = 2000


def _combine_body(x_ref, root_ref, bias_ref, p0_ref, p1_ref, out_ref):
    out_ref[...] = (
        jnp.dot(x_ref[...], root_ref[...], preferred_element_type=jnp.float32)
        + bias_ref[...] + p0_ref[...] + p1_ref[...])


def _combine(x, root, bias2d, partials):
    nblk = N_NODES // _CBLK
    return pl.pallas_call(
        _combine_body,
        grid=(nblk,),
        in_specs=[
            pl.BlockSpec((_CBLK, D), lambda i: (i, 0)),
            pl.BlockSpec((D, D), lambda i: (0, 0)),
            pl.BlockSpec((1, D), lambda i: (0, 0)),
            pl.BlockSpec((_CBLK, D), lambda i: (i, 0)),
            pl.BlockSpec((_CBLK, D), lambda i, _n=nblk: (i + _n, 0)),
        ],
        out_specs=pl.BlockSpec((_CBLK, D), lambda i: (i, 0)),
        out_shape=jax.ShapeDtypeStruct((N_NODES, D), jnp.float32),
    )(x, root, bias2d, partials, partials)


def kernel(x, edge_index, edge_type, basis, att, root, bias):
    src = edge_index[0].astype(jnp.int32)
    dst = edge_index[1].astype(jnp.int32)
    et = edge_type.astype(jnp.int32)
    flat_idx = _flat_idx(et.reshape(-1, D), src.reshape(-1, D)).reshape(-1)
    xw = _xw_table(att, x, basis)
    zeros = jnp.zeros((ROWS_PER_SUB, D), jnp.float32)
    partials = _sc_aggregate(xw, flat_idx, dst, zeros)
    return _combine(x, root, bias.reshape(1, D), partials)
